# Initial kernel scaffold; baseline (speedup 1.0000x reference)
import jax
import jax.numpy as jnp
from jax.experimental import pallas as pl


def kernel(node_feat, pin_feat, edge_attr, Wl, bl, Wr, br, att, gat_bias, Wsl, bsl, Wsr, edge_index):
    raise NotImplementedError("write your pallas kernel here")


# speedup vs baseline: 10.5060x; 10.5060x over previous
import functools

import jax
import jax.numpy as jnp
from jax import lax
from jax.experimental import pallas as pl
from jax.experimental.pallas import tpu as pltpu
from jax.experimental.pallas import tpu_sc as plsc

F32 = jnp.float32
I32 = jnp.int32

NC = 2
NS = 16
NW = NC * NS

CB = 256
JB = CB // 128

NP_ = 1600000
NN_ = 100000
NH_ = 100000
NCH = NP_ // CB
TK_P = -(-NCH // NW)
NCH_H = -(-NH_ // CB)
HPAD = NCH_H * CB
TK_H = -(-NCH_H // NW)
TK_HZ = -(-NCH_H // NS)

O_WL, O_BL, O_WR, O_BR, O_ATT, O_GB, O_WSL, O_BSL, O_WSR = (
    0, 128, 144, 160, 176, 192, 208, 464, 480)
NPARAMS = 608

_CPARAMS = dict(needs_layout_passes=False, use_tc_tiling_on_sc=False)


def _mesh():
    return plsc.VectorSubcoreMesh(
        core_axis_name="c", subcore_axis_name="s",
        num_cores=NC, num_subcores=NS)


def _leaky(x, s):
    return jnp.where(x >= 0, x, s * x)


def _iota16():
    return lax.iota(I32, 16)


def _full16(v):
    return jnp.full((16,), v, I32)


def _wid():
    return lax.axis_index("c") * NS + lax.axis_index("s")


def _xl_from_soa(nfgb, pb, g, p0v, p1v, off_w, off_b=None):
    nfk = [nfgb[pl.ds(k * CB + g * 16, 16)] for k in range(6)]
    out = []
    for ch in range(16):
        acc = (p0v * pb[off_w + 6 * 16 + ch, :]
               + p1v * pb[off_w + 7 * 16 + ch, :])
        if off_b is not None:
            acc = acc + pb[off_b + ch, :]
        for k in range(6):
            acc = acc + nfk[k] * pb[off_w + k * 16 + ch, :]
        out.append(acc)
    return out


def _logit(xl, eav, pb):
    lg = None
    for ch in range(16):
        z = _leaky(xl[ch] + (eav * pb[O_WR + ch, :] + pb[O_BR + ch, :]), 0.2)
        t = pb[O_ATT + ch, :] * z
        lg = t if lg is None else lg + t
    return lg


def _k1_body(nf8, p0, p1, src, he, ea, params,
             nfg_o, lg_o, tmax_o,
             ea_v, pvm, srcb, heb, p0b, p1b, rows, soa, lgb, rmx, sem):
    wid = _wid()
    pltpu.sync_copy(params, pvm)
    pltpu.sync_copy(ea, ea_v)
    rmx[...] = jnp.full((16,), -1e30, F32)

    def chunk(t, _):
        c = t * NW + wid

        @pl.when(c < NCH)
        def _():
            base = c * CB
            for j in range(JB):
                pltpu.sync_copy(src.at[pl.ds(base + j * 128, 128)], srcb.at[j])
            pltpu.sync_copy(he.at[pl.ds(base, CB)], heb)
            pltpu.sync_copy(p0.at[pl.ds(base, CB)], p0b)
            pltpu.sync_copy(p1.at[pl.ds(base, CB)], p1b)
            descs = [pltpu.async_copy(nf8.at[srcb.at[j]],
                                      rows.at[pl.ds(j * 128, 128), :], sem)
                     for j in range(JB)]
            for d in descs:
                d.wait()

            def grp(g, _):
                s16 = pl.ds(g * 16, 16)
                pin16 = g * 16 + _iota16()
                p0v = p0b[s16]
                p1v = p1b[s16]
                nfk = [plsc.load_gather(rows, [pin16, _full16(k)])
                       for k in range(6)]
                xl = []
                for ch in range(16):
                    acc = (pvm[O_BL + ch, :] + p0v * pvm[O_WL + 6 * 16 + ch, :]
                           + p1v * pvm[O_WL + 7 * 16 + ch, :])
                    for k in range(6):
                        acc = acc + nfk[k] * pvm[O_WL + k * 16 + ch, :]
                    xl.append(acc)
                for k in range(6):
                    soa[pl.ds(k * CB + g * 16, 16)] = nfk[k]
                hev = heb[s16]
                eav = plsc.load_gather(ea_v, [hev])
                lg = _logit(xl, eav, pvm)
                lg = jnp.where(hev == base + pin16, -1e30, lg)
                lgb[s16] = lg
                rmx[...] = jnp.maximum(rmx[...], lg)

                @pl.when(base < NH_)
                def _():
                    eas = ea_v[pl.ds(base + g * 16, 16)]
                    lsl = _logit(xl, eas, pvm)
                    lsl = jnp.where(base + pin16 < NH_, lsl, -1e30)
                    rmx[...] = jnp.maximum(rmx[...], lsl)

                return 0

            lax.fori_loop(0, CB // 16, grp, 0)
            pltpu.sync_copy(lgb, lg_o.at[pl.ds(base, CB)])
            pltpu.sync_copy(soa, nfg_o.at[c])

        return 0

    lax.fori_loop(0, TK_P, chunk, 0)
    pltpu.sync_copy(rmx, tmax_o.at[pl.ds(wid * 16, 16)])


def _k2_body(nfg, p0, p1, he, lgs, mv, params, z16, z1,
             nump_o, denp_o,
             num_sh, den_sh, pvm, mb, nfgb, p0b, p1b, he2, lgb, ebuf,
             exl, sem):
    core = lax.axis_index("c")
    sub = lax.axis_index("s")
    wid = core * NS + sub
    pltpu.sync_copy(params, pvm)
    pltpu.sync_copy(mv, mb)

    def zchunk(t, _):
        c = t * NS + sub

        @pl.when(c < NCH_H)
        def _():
            r = pl.ds(c * CB, CB)
            pltpu.sync_copy(z16.at[r, :], num_sh.at[r, :])
            pltpu.sync_copy(z1.at[r], den_sh.at[r])

        return 0

    lax.fori_loop(0, TK_HZ, zchunk, 0)
    plsc.subcore_barrier()

    def chunk(t, _):
        c = t * NW + wid

        @pl.when(c < NCH)
        def _():
            base = c * CB
            m = mb[...]
            pltpu.sync_copy(nfg.at[c], nfgb)
            pltpu.sync_copy(p0.at[pl.ds(base, CB)], p0b)
            pltpu.sync_copy(p1.at[pl.ds(base, CB)], p1b)
            pltpu.sync_copy(lgs.at[pl.ds(base, CB)], lgb)
            for j in range(JB):
                pltpu.sync_copy(he.at[pl.ds(base + j * 128, 128)], he2.at[j])

            def grp(g, _):
                s16 = pl.ds(g * 16, 16)
                pin16 = g * 16 + _iota16()
                xl = _xl_from_soa(nfgb, pvm, g, p0b[s16], p1b[s16],
                                  O_WL, O_BL)
                e = jnp.exp(lgb[s16] - m)
                ebuf[s16] = e
                for ch in range(16):
                    plsc.store_scatter(exl, [pin16, _full16(ch)], e * xl[ch])

                return 0

            lax.fori_loop(0, CB // 16, grp, 0)
            for j in range(JB):
                pltpu.sync_copy(exl.at[pl.ds(j * 128, 128), :],
                                num_sh.at[he2.at[j]], add=True)
                pltpu.sync_copy(ebuf.at[pl.ds(j * 128, 128)],
                                den_sh.at[he2.at[j]], add=True)

        return 0

    lax.fori_loop(0, TK_P, chunk, 0)
    plsc.subcore_barrier()

    def wchunk(t, _):
        c = t * NS + sub

        @pl.when(c < NCH_H)
        def _():
            r = pl.ds(c * CB, CB)

            @pl.when(core == 0)
            def _():
                pltpu.sync_copy(num_sh.at[r, :], nump_o.at[0, r, :])
                pltpu.sync_copy(den_sh.at[r], denp_o.at[0, r])

            @pl.when(core == 1)
            def _():
                pltpu.sync_copy(num_sh.at[r, :], nump_o.at[1, r, :])
                pltpu.sync_copy(den_sh.at[r], denp_o.at[1, r])

        return 0

    lax.fori_loop(0, TK_HZ, wchunk, 0)


def _k3_body(nump, denp, nfg, p0, p1, ea, mv, params,
             g_o,
             pvm, mb, n0, n1, d0, d1, nfgb, p0b, p1b, eab, gb, sem):
    wid = _wid()
    pltpu.sync_copy(params, pvm)
    pltpu.sync_copy(mv, mb)

    def chunk(t, _):
        c = t * NW + wid

        @pl.when(c < NCH_H)
        def _():
            base = c * CB
            r = pl.ds(base, CB)
            m = mb[...]
            pltpu.sync_copy(nump.at[0, r, :], n0)
            pltpu.sync_copy(nump.at[1, r, :], n1)
            pltpu.sync_copy(denp.at[0, r], d0)
            pltpu.sync_copy(denp.at[1, r], d1)
            pltpu.sync_copy(nfg.at[c], nfgb)
            pltpu.sync_copy(p0.at[r], p0b)
            pltpu.sync_copy(p1.at[r], p1b)
            pltpu.sync_copy(ea.at[r], eab)

            def grp(g, _):
                s16 = pl.ds(g * 16, 16)
                pin16 = g * 16 + _iota16()
                xlc = _xl_from_soa(nfgb, pvm, g, p0b[s16], p1b[s16],
                                   O_WL, O_BL)
                eav = eab[s16]
                lsl = _logit(xlc, eav, pvm)
                esl = jnp.exp(lsl - m)
                den = d0[s16] + d1[s16] + esl
                ef = []
                for ch in range(16):
                    nu = (plsc.load_gather(n0, [pin16, _full16(ch)])
                          + plsc.load_gather(n1, [pin16, _full16(ch)])
                          + esl * xlc[ch])
                    ef.append(_leaky(nu / den + pvm[O_GB + ch, :], 0.1))
                for ch in range(16):
                    acc = pvm[O_BSL + ch, :]
                    for k in range(16):
                        acc = acc + ef[k] * pvm[O_WSL + k * 16 + ch, :]
                    plsc.store_scatter(gb, [pin16, _full16(ch)], acc)
                return 0

            lax.fori_loop(0, CB // 16, grp, 0)
            pltpu.sync_copy(gb, g_o.at[r, :])

        return 0

    lax.fori_loop(0, TK_H, chunk, 0)


def _k4_body(nfg, p0, p1, he, src, g_in, params, z16, z1,
             pf_o, sump_o, cntp_o,
             s_sh, c_sh, pvm, nfgb, p0b, p1b, he2, sr2, grows, pob, ones,
             sem):
    core = lax.axis_index("c")
    sub = lax.axis_index("s")
    wid = core * NS + sub
    pltpu.sync_copy(params, pvm)
    for i in range(128 // 16):
        ones[pl.ds(i * 16, 16)] = jnp.ones((16,), F32)

    def zchunk(t, _):
        c = t * NS + sub

        @pl.when(c < NCH_H)
        def _():
            r = pl.ds(c * CB, CB)
            pltpu.sync_copy(z16.at[r, :], s_sh.at[r, :])
            pltpu.sync_copy(z1.at[r], c_sh.at[r])

        return 0

    lax.fori_loop(0, TK_HZ, zchunk, 0)
    plsc.subcore_barrier()

    def chunk(t, _):
        c = t * NW + wid

        @pl.when(c < NCH)
        def _():
            base = c * CB
            pltpu.sync_copy(nfg.at[c], nfgb)
            pltpu.sync_copy(p0.at[pl.ds(base, CB)], p0b)
            pltpu.sync_copy(p1.at[pl.ds(base, CB)], p1b)
            for j in range(JB):
                pltpu.sync_copy(he.at[pl.ds(base + j * 128, 128)], he2.at[j])
                pltpu.sync_copy(src.at[pl.ds(base + j * 128, 128)], sr2.at[j])
            descs = [pltpu.async_copy(g_in.at[he2.at[j]],
                                      grows.at[pl.ds(j * 128, 128), :], sem)
                     for j in range(JB)]
            for d in descs:
                d.wait()

            def grp(g, _):
                s16 = pl.ds(g * 16, 16)
                pin16 = g * 16 + _iota16()
                xv = _xl_from_soa(nfgb, pvm, g, p0b[s16], p1b[s16], 0)
                for ch in range(16):
                    grc = plsc.load_gather(grows, [pin16, _full16(ch)])
                    po = _leaky(grc + xv[ch], 0.1)
                    plsc.store_scatter(pob, [pin16, _full16(ch)], po)
                return 0

            lax.fori_loop(0, CB // 16, grp, 0)
            pltpu.sync_copy(pob, pf_o.at[pl.ds(base, CB), :])
            for j in range(JB):
                pltpu.sync_copy(pob.at[pl.ds(j * 128, 128), :],
                                s_sh.at[sr2.at[j]], add=True)
                pltpu.sync_copy(ones, c_sh.at[sr2.at[j]], add=True)

        return 0

    lax.fori_loop(0, TK_P, chunk, 0)
    plsc.subcore_barrier()

    def wchunk(t, _):
        c = t * NS + sub

        @pl.when(c < NCH_H)
        def _():
            r = pl.ds(c * CB, CB)

            @pl.when(core == 0)
            def _():
                pltpu.sync_copy(s_sh.at[r, :], sump_o.at[0, r, :])
                pltpu.sync_copy(c_sh.at[r], cntp_o.at[0, r])

            @pl.when(core == 1)
            def _():
                pltpu.sync_copy(s_sh.at[r, :], sump_o.at[1, r, :])
                pltpu.sync_copy(c_sh.at[r], cntp_o.at[1, r])

        return 0

    lax.fori_loop(0, TK_HZ, wchunk, 0)


def _k5_body(sump, cntp, nf_o, s0, s1, c0, c1, ob, sem):
    wid = _wid()

    def chunk(t, _):
        c = t * NW + wid

        @pl.when(c < NCH_H)
        def _():
            r = pl.ds(c * CB, CB)
            pltpu.sync_copy(sump.at[0, r, :], s0)
            pltpu.sync_copy(sump.at[1, r, :], s1)
            pltpu.sync_copy(cntp.at[0, r], c0)
            pltpu.sync_copy(cntp.at[1, r], c1)

            def grp(g, _):
                s16 = pl.ds(g * 16, 16)
                pin16 = g * 16 + _iota16()
                cnt = jnp.maximum(c0[s16] + c1[s16], 1.0)
                for ch in range(16):
                    sv = (plsc.load_gather(s0, [pin16, _full16(ch)])
                          + plsc.load_gather(s1, [pin16, _full16(ch)]))
                    plsc.store_scatter(ob, [pin16, _full16(ch)], sv / cnt)
                return 0

            lax.fori_loop(0, CB // 16, grp, 0)
            pltpu.sync_copy(ob, nf_o.at[r, :])

        return 0

    lax.fori_loop(0, TK_H, chunk, 0)


def _sds(shape, dtype=F32):
    return jax.ShapeDtypeStruct(shape, dtype)


@functools.lru_cache(maxsize=None)
def _k1():
  return pl.kernel(
    _k1_body,
    out_type=(_sds((NCH, 6 * CB)), _sds((NP_,)), _sds((NW * 16,))),
    mesh=_mesh(),
    compiler_params=pltpu.CompilerParams(**_CPARAMS),
    scratch_types=[
        pltpu.VMEM((HPAD,), F32),
        pltpu.VMEM((NPARAMS, 16), F32),
        pltpu.VMEM((JB, 128), I32),
        pltpu.VMEM((CB,), I32),
        pltpu.VMEM((CB,), F32),
        pltpu.VMEM((CB,), F32),
        pltpu.VMEM((CB, 8), F32),
        pltpu.VMEM((6 * CB,), F32),
        pltpu.VMEM((CB,), F32),
        pltpu.VMEM((16,), F32),
        pltpu.SemaphoreType.DMA,
    ])


@functools.lru_cache(maxsize=None)
def _k2():
  return pl.kernel(
    _k2_body,
    out_type=(_sds((NC, HPAD, 16)), _sds((NC, HPAD))),
    mesh=_mesh(),
    compiler_params=pltpu.CompilerParams(**_CPARAMS),
    scratch_types=[
        pltpu.VMEM_SHARED((HPAD, 16), F32),
        pltpu.VMEM_SHARED((HPAD,), F32),
        pltpu.VMEM((144, 16), F32),
        pltpu.VMEM((16,), F32),
        pltpu.VMEM((6 * CB,), F32),
        pltpu.VMEM((CB,), F32),
        pltpu.VMEM((CB,), F32),
        pltpu.VMEM((JB, 128), I32),
        pltpu.VMEM((CB,), F32),
        pltpu.VMEM((CB,), F32),
        pltpu.VMEM((CB, 16), F32),
        pltpu.SemaphoreType.DMA,
    ])


@functools.lru_cache(maxsize=None)
def _k3():
  return pl.kernel(
    _k3_body,
    out_type=_sds((HPAD, 16)),
    mesh=_mesh(),
    compiler_params=pltpu.CompilerParams(**_CPARAMS),
    scratch_types=[
        pltpu.VMEM((NPARAMS, 16), F32),
        pltpu.VMEM((16,), F32),
        pltpu.VMEM((CB, 16), F32),
        pltpu.VMEM((CB, 16), F32),
        pltpu.VMEM((CB,), F32),
        pltpu.VMEM((CB,), F32),
        pltpu.VMEM((6 * CB,), F32),
        pltpu.VMEM((CB,), F32),
        pltpu.VMEM((CB,), F32),
        pltpu.VMEM((CB,), F32),
        pltpu.VMEM((CB, 16), F32),
        pltpu.SemaphoreType.DMA,
    ])


@functools.lru_cache(maxsize=None)
def _k4():
  return pl.kernel(
    _k4_body,
    out_type=(_sds((NP_, 16)), _sds((NC, HPAD, 16)), _sds((NC, HPAD))),
    mesh=_mesh(),
    compiler_params=pltpu.CompilerParams(**_CPARAMS),
    scratch_types=[
        pltpu.VMEM_SHARED((HPAD, 16), F32),
        pltpu.VMEM_SHARED((HPAD,), F32),
        pltpu.VMEM((128, 16), F32),
        pltpu.VMEM((6 * CB,), F32),
        pltpu.VMEM((CB,), F32),
        pltpu.VMEM((CB,), F32),
        pltpu.VMEM((JB, 128), I32),
        pltpu.VMEM((JB, 128), I32),
        pltpu.VMEM((CB, 16), F32),
        pltpu.VMEM((CB, 16), F32),
        pltpu.VMEM((128,), F32),
        pltpu.SemaphoreType.DMA,
    ])


@functools.lru_cache(maxsize=None)
def _k5():
  return pl.kernel(
    _k5_body,
    out_type=_sds((HPAD, 16)),
    mesh=_mesh(),
    compiler_params=pltpu.CompilerParams(**_CPARAMS),
    scratch_types=[
        pltpu.VMEM((CB, 16), F32),
        pltpu.VMEM((CB, 16), F32),
        pltpu.VMEM((CB,), F32),
        pltpu.VMEM((CB,), F32),
        pltpu.VMEM((CB, 16), F32),
        pltpu.SemaphoreType.DMA,
    ])


def kernel(node_feat, pin_feat, edge_attr, Wl, bl, Wr, br, att, gat_bias,
           Wsl, bsl, Wsr, edge_index):
    nf8 = jnp.pad(node_feat, ((0, 0), (0, 2)))
    p0 = pin_feat[:, 0] + 0.0
    p1 = pin_feat[:, 1] + 0.0
    src = edge_index[0]
    he = edge_index[1]
    ea = jnp.pad(edge_attr[:, 0], (0, HPAD - NH_))
    params = jnp.concatenate([
        Wl.reshape(-1), bl, Wr.reshape(-1), br, att, gat_bias,
        Wsl.reshape(-1), bsl, Wsr.reshape(-1)]).astype(F32)
    paramsb = jnp.tile(params[:, None], (1, 16))

    nfg, lgs, tmax = _k1()(nf8, p0, p1, src, he, ea, paramsb)
    mv = jnp.full((16,), jnp.max(tmax), F32)
    z16 = jnp.zeros((HPAD, 16), F32)
    z1 = jnp.zeros((HPAD,), F32)
    nump, denp = _k2()(nfg, p0, p1, he, lgs, mv, paramsb[:144], z16, z1)
    g = _k3()(nump, denp, nfg, p0, p1, ea, mv, paramsb)
    pf_out, sump, cntp = _k4()(nfg, p0, p1, he, src, g,
                               paramsb[O_WSR:O_WSR + 128], z16, z1)
    nf_pad = _k5()(sump, cntp)
    return nf_pad[:NN_], pf_out
